# baseline (speedup 1.0000x reference)
import functools

import jax
import jax.numpy as jnp
from jax.experimental import pallas as pl
from jax.experimental.pallas import tpu as pltpu


def _se_kernel(xa_ref, xb_ref, w1t_ref, b1_ref, w2t_ref, b2_ref, o_ref,
               *, inv_hw):
    xa = xa_ref[0]
    xb = xb_ref[0]
    nb = xa.shape[0]

    sa = jnp.sum(xa, axis=-1)
    sb = jnp.sum(xb, axis=-1)
    s = jnp.concatenate([sa, sb], axis=0) * inv_hw

    h = jnp.dot(s, w1t_ref[...], preferred_element_type=jnp.float32)
    h = jnp.maximum(h + b1_ref[...], 0.0)
    g = jnp.dot(h, w2t_ref[...], preferred_element_type=jnp.float32)
    g = jax.nn.sigmoid(g + b2_ref[...])

    o_ref[0] = xa * g[:nb, :, None]
    o_ref[1] = xb * g[nb:, :, None]


@jax.jit
def _se_forward(x_nchw, w1, b1, w2, b2):
    n, c, h, w = x_nchw.shape
    cmid = w1.shape[0]
    hw = h * w
    half = n // 2

    x4 = x_nchw.reshape(2, half, c, hw)
    w1t = w1.T
    w2t = w2.T
    b1r = b1.reshape(1, cmid)
    b2r = b2.reshape(1, c)

    nb = 16
    while nb > 1 and half % nb:
        nb //= 2
    grid = (half // nb,)

    out4 = pl.pallas_call(
        functools.partial(_se_kernel, inv_hw=1.0 / hw),
        out_shape=jax.ShapeDtypeStruct((2, half, c, hw), x4.dtype),
        grid_spec=pl.GridSpec(
            grid=grid,
            in_specs=[
                pl.BlockSpec((1, nb, c, hw), lambda i: (0, i, 0, 0)),
                pl.BlockSpec((1, nb, c, hw), lambda i: (1, i, 0, 0)),
                pl.BlockSpec((c, cmid), lambda i: (0, 0)),
                pl.BlockSpec((1, cmid), lambda i: (0, 0)),
                pl.BlockSpec((cmid, c), lambda i: (0, 0)),
                pl.BlockSpec((1, c), lambda i: (0, 0)),
            ],
            out_specs=pl.BlockSpec((2, nb, c, hw), lambda i: (0, i, 0, 0)),
        ),
        compiler_params=pltpu.CompilerParams(
            dimension_semantics=("parallel",),
            vmem_limit_bytes=60 << 20,
        ),
    )(x4, x4, w1t, b1r, w2t, b2r)
    return out4.reshape(n, c, h, w)


def kernel(x_nchw, w1, b1, w2, b2):
    return _se_forward(x_nchw, w1, b1, w2, b2)


# speedup vs baseline: 1.0006x; 1.0006x over previous
import functools

import jax
import jax.numpy as jnp
from jax.experimental import pallas as pl
from jax.experimental.pallas import tpu as pltpu


def _se_kernel(x_ref, w1t_ref, b1_ref, w2t_ref, b2_ref, o_ref, *, inv_hw):
    x = x_ref[...]

    s = jnp.sum(x, axis=-1) * inv_hw
    h = jnp.dot(s, w1t_ref[...], preferred_element_type=jnp.float32)
    h = jnp.maximum(h + b1_ref[...], 0.0)
    g = jnp.dot(h, w2t_ref[...], preferred_element_type=jnp.float32)
    g = jax.nn.sigmoid(g + b2_ref[...])

    o_ref[...] = x * g[:, :, None]


@jax.jit
def _se_forward(x_nchw, w1, b1, w2, b2):
    n, c, h, w = x_nchw.shape
    cmid = w1.shape[0]
    hw = h * w

    x3 = x_nchw.reshape(n, c, hw)
    w1t = w1.T
    w2t = w2.T
    b1r = b1.reshape(1, cmid)
    b2r = b2.reshape(1, c)

    nb = 16
    while nb > 1 and n % nb:
        nb //= 2
    grid = (n // nb,)

    out3 = pl.pallas_call(
        functools.partial(_se_kernel, inv_hw=1.0 / hw),
        out_shape=jax.ShapeDtypeStruct((n, c, hw), x3.dtype),
        grid_spec=pl.GridSpec(
            grid=grid,
            in_specs=[
                pl.BlockSpec((nb, c, hw), lambda i: (i, 0, 0)),
                pl.BlockSpec((c, cmid), lambda i: (0, 0)),
                pl.BlockSpec((1, cmid), lambda i: (0, 0)),
                pl.BlockSpec((cmid, c), lambda i: (0, 0)),
                pl.BlockSpec((1, c), lambda i: (0, 0)),
            ],
            out_specs=pl.BlockSpec((nb, c, hw), lambda i: (i, 0, 0)),
        ),
        compiler_params=pltpu.CompilerParams(
            dimension_semantics=("parallel",),
            vmem_limit_bytes=60 << 20,
        ),
    )(x3, w1t, b1r, w2t, b2r)
    return out3.reshape(n, c, h, w)


def kernel(x_nchw, w1, b1, w2, b2):
    return _se_forward(x_nchw, w1, b1, w2, b2)


# speedup vs baseline: 1.0078x; 1.0072x over previous
import functools

import jax
import jax.numpy as jnp
from jax.experimental import pallas as pl
from jax.experimental.pallas import tpu as pltpu


def _se_kernel(x_ref, w1t_ref, b1_ref, w2t_ref, b2_ref, o_ref, *, inv_hw):
    x = x_ref[...]

    s = jnp.sum(x, axis=-1) * inv_hw
    h = jnp.dot(s, w1t_ref[...], preferred_element_type=jnp.float32)
    h = jnp.maximum(h + b1_ref[...], 0.0)
    g = jnp.dot(h, w2t_ref[...], preferred_element_type=jnp.float32)
    g = jax.nn.sigmoid(g + b2_ref[...])

    o_ref[...] = x * g[:, :, None]


@jax.jit
def _se_forward(x_nchw, w1, b1, w2, b2):
    n, c, h, w = x_nchw.shape
    cmid = w1.shape[0]
    hw = h * w

    x3 = x_nchw.reshape(n, c, hw)
    w1t = w1.T
    w2t = w2.T
    b1r = b1.reshape(1, cmid)
    b2r = b2.reshape(1, c)

    nb = 32
    while nb > 1 and n % nb:
        nb //= 2
    grid = (n // nb,)

    out3 = pl.pallas_call(
        functools.partial(_se_kernel, inv_hw=1.0 / hw),
        out_shape=jax.ShapeDtypeStruct((n, c, hw), x3.dtype),
        grid_spec=pl.GridSpec(
            grid=grid,
            in_specs=[
                pl.BlockSpec((nb, c, hw), lambda i: (i, 0, 0)),
                pl.BlockSpec((c, cmid), lambda i: (0, 0)),
                pl.BlockSpec((1, cmid), lambda i: (0, 0)),
                pl.BlockSpec((cmid, c), lambda i: (0, 0)),
                pl.BlockSpec((1, c), lambda i: (0, 0)),
            ],
            out_specs=pl.BlockSpec((nb, c, hw), lambda i: (i, 0, 0)),
        ),
        compiler_params=pltpu.CompilerParams(
            dimension_semantics=("parallel",),
            vmem_limit_bytes=60 << 20,
        ),
    )(x3, w1t, b1r, w2t, b2r)
    return out3.reshape(n, c, h, w)


def kernel(x_nchw, w1, b1, w2, b2):
    return _se_forward(x_nchw, w1, b1, w2, b2)
